# baseline (speedup 1.0000x reference)
import jax
import jax.numpy as jnp
from jax.experimental import pallas as pl
from jax.experimental.pallas import tpu as pltpu

_NC = 80
_MASK = 32
_NF = 5 + _NC + _MASK
_COLS = 5 + _NC
_MAXDET = 300
_IOU_T = 0.45
_CONF_T = 0.25
_NEG = -1e9
_N = 20000
_LANES = 128
_ROWS = 160
_NPAD = _ROWS * _LANES
_FULL = _N // _LANES
_TAIL = _N - _FULL * _LANES
_UNROLL = 6


def _eye(nrows):
    return (jax.lax.broadcasted_iota(jnp.int32, (nrows, _LANES), 0)
            == jax.lax.broadcasted_iota(jnp.int32, (nrows, _LANES), 1)
            ).astype(jnp.float32)


def _nms_body(rows_ref, ob_ref, om_ref,
              y1_scr, x1_scr, y2_scr, x2_scr, ar_scr, key_scr, s_scr):
    lane = jax.lax.broadcasted_iota(jnp.int32, (1, _LANES), 1)
    c_iota = jax.lax.broadcasted_iota(jnp.int32, (_NC, _LANES), 0)

    def block(r, nrows, eye):
        tile = rows_ref[0, pl.ds(r * _LANES, nrows), 0:_COLS]
        tt = jax.lax.dot_general(
            tile, eye, (((0,), (0,)), ((), ())),
            preferred_element_type=jnp.float32,
            precision=jax.lax.Precision.HIGHEST)
        obj = tt[4:5, :]
        cls = tt[5:5 + _NC, :] * obj
        m = jnp.max(cls, axis=0, keepdims=True)
        ci = jnp.min(jnp.where(cls == m, c_iota, _NC),
                     axis=0, keepdims=True)
        s = jnp.where(obj > _CONF_T, m, _NEG)
        if nrows < _LANES:
            s = jnp.where(lane < nrows, s, _NEG)
        xc = tt[0:1, :]
        yc = tt[1:2, :]
        w2 = tt[2:3, :] * 0.5
        h2 = tt[3:4, :] * 0.5
        y1 = yc - h2
        x1 = xc - w2
        y2 = yc + h2
        x2 = xc + w2
        s_scr[pl.ds(r, 1), :] = s
        y1_scr[pl.ds(r, 1), :] = y1
        x1_scr[pl.ds(r, 1), :] = x1
        y2_scr[pl.ds(r, 1), :] = y2
        x2_scr[pl.ds(r, 1), :] = x2
        ar_scr[pl.ds(r, 1), :] = (y2 - y1) * (x2 - x1)
        key_scr[pl.ds(r, 1), :] = (r * _LANES + lane) * 128 + ci

    eye128 = _eye(_LANES)

    def p1_body(g, _):
        for j in range(_UNROLL):
            block(g * _UNROLL + j, _LANES, eye128)
        return 0
    jax.lax.fori_loop(0, _FULL // _UNROLL, p1_body, 0)
    block(_FULL, _TAIL, _eye(_TAIL))

    ztail = jnp.zeros((_ROWS - _FULL - 1, _LANES), jnp.float32)
    s_scr[pl.ds(_FULL + 1, _ROWS - _FULL - 1), :] = ztail + _NEG
    y1_scr[pl.ds(_FULL + 1, _ROWS - _FULL - 1), :] = ztail
    x1_scr[pl.ds(_FULL + 1, _ROWS - _FULL - 1), :] = ztail
    y2_scr[pl.ds(_FULL + 1, _ROWS - _FULL - 1), :] = ztail
    x2_scr[pl.ds(_FULL + 1, _ROWS - _FULL - 1), :] = ztail
    ar_scr[pl.ds(_FULL + 1, _ROWS - _FULL - 1), :] = ztail
    key_scr[pl.ds(_FULL + 1, _ROWS - _FULL - 1), :] = ztail.astype(jnp.int32)

    keyp_all = key_scr[...]

    def pick(sv):
        best = jnp.max(sv)
        key = jnp.min(jnp.where(sv == best, key_scr[...], _NPAD * 128))
        return best, key

    def corners(key):
        row = rows_ref[0, pl.ds(key >> 7, 1), 0:4]
        bx = row[:, 0:1]
        by = row[:, 1:2]
        w2 = row[:, 2:3] * 0.5
        h2 = row[:, 3:4] * 0.5
        return by - h2, bx - w2, by + h2, bx + w2

    def body(i, carry):
        s, ok, sb, sk, s1, s2, s3, s4 = carry

        def fallback(_):
            b, k = pick(s)
            return (b, k) + corners(k)

        best, key, by1, bx1, by2, bx2 = jax.lax.cond(
            ok != 0.0, lambda _: (sb, sk, s1, s2, s3, s4), fallback, None)

        s_excl = jnp.where(key_scr[...] == key, _NEG, s)
        yy1 = jnp.maximum(y1_scr[...], by1)
        xx1 = jnp.maximum(x1_scr[...], bx1)
        yy2 = jnp.minimum(y2_scr[...], by2)
        xx2 = jnp.minimum(x2_scr[...], bx2)
        inter = (jnp.clip(yy2 - yy1, 0.0) * jnp.clip(xx2 - xx1, 0.0))
        barea = (by2 - by1) * (bx2 - bx1)
        union = ar_scr[...] + barea - inter + 1e-9
        s_next = jnp.where(inter > _IOU_T * union, _NEG, s_excl)

        nb, nk = pick(s_excl)
        ny1, nx1, ny2, nx2 = corners(nk)
        qy1 = jnp.maximum(ny1, by1)
        qx1 = jnp.maximum(nx1, bx1)
        qy2 = jnp.minimum(ny2, by2)
        qx2 = jnp.minimum(nx2, bx2)
        qi = (jnp.clip(qy2 - qy1, 0.0) * jnp.clip(qx2 - qx1, 0.0))
        qa = (ny2 - ny1) * (nx2 - nx1)
        qu = qa + barea - qi + 1e-9
        ok_next = jnp.where(qi > _IOU_T * qu, 0.0, 1.0)[0, 0]

        idx = key >> 7
        cls = key & 127
        valid = best > _NEG * 0.5
        main = jnp.concatenate(
            [by1, bx1, by2, bx2,
             jnp.broadcast_to(cls.astype(jnp.float32), (1, 1)),
             jnp.broadcast_to(best, (1, 1)),
             jnp.zeros((1, 2), jnp.float32)], axis=1)
        ob_ref[pl.ds(i, 1), :] = jnp.where(valid, main, 0.0)
        mrow = rows_ref[0, pl.ds(idx, 1), 5 + _NC:]
        om_ref[pl.ds(i, 1), :] = jnp.where(valid, mrow, 0.0)
        return (s_next, ok_next, nb, nk, ny1, nx1, ny2, nx2)

    z11 = jnp.zeros((1, 1), jnp.float32)
    jax.lax.fori_loop(
        0, _MAXDET, body,
        (s_scr[...], jnp.float32(0), jnp.float32(0), jnp.int32(0),
         z11, z11, z11, z11))


@jax.jit
def kernel(predictions):
    out_shapes = (
        jax.ShapeDtypeStruct((_MAXDET, 8), jnp.float32),
        jax.ShapeDtypeStruct((_MAXDET, _MASK), jnp.float32),
    )
    main, masks = pl.pallas_call(
        _nms_body,
        out_shape=out_shapes,
        scratch_shapes=([pltpu.VMEM((_ROWS, _LANES), jnp.float32)
                         for _ in range(5)]
                        + [pltpu.VMEM((_ROWS, _LANES), jnp.int32)]
                        + [pltpu.VMEM((_ROWS, _LANES), jnp.float32)]),
    )(predictions)
    return (main[None, :, :4],
            main[:, 4].reshape(1, _MAXDET),
            main[:, 5].reshape(1, _MAXDET),
            masks[None])


# speedup vs baseline: 1.3806x; 1.3806x over previous
import jax
import jax.numpy as jnp
from jax.experimental import pallas as pl
from jax.experimental.pallas import tpu as pltpu

_NC = 80
_MASK = 32
_NF = 5 + _NC + _MASK
_COLS = 5 + _NC
_MAXDET = 300
_IOU_T = 0.45
_CONF_T = 0.25
_NEG = -1e9
_N = 20000
_LANES = 128
_ROWS = 160
_NPAD = _ROWS * _LANES
_FULL = _N // _LANES
_TAIL = _N - _FULL * _LANES
_UNROLL = 6


def _eye(nrows):
    return (jax.lax.broadcasted_iota(jnp.int32, (nrows, _LANES), 0)
            == jax.lax.broadcasted_iota(jnp.int32, (nrows, _LANES), 1)
            ).astype(jnp.float32)


def _nms_body(rows_ref, ob_ref, om_ref,
              y1_scr, x1_scr, y2_scr, x2_scr, ar_scr, key_scr, s_scr):
    lane = jax.lax.broadcasted_iota(jnp.int32, (1, _LANES), 1)
    c_iota = jax.lax.broadcasted_iota(jnp.int32, (_NC, _LANES), 0)

    def block(r, nrows, eye):
        tile = rows_ref[0, pl.ds(r * _LANES, nrows), 0:_COLS]
        tt = jax.lax.dot_general(
            tile, eye, (((0,), (0,)), ((), ())),
            preferred_element_type=jnp.float32,
            precision=jax.lax.Precision.HIGHEST)
        obj = tt[4:5, :]
        cls = tt[5:5 + _NC, :] * obj
        m = jnp.max(cls, axis=0, keepdims=True)
        ci = jnp.min(jnp.where(cls == m, c_iota, _NC),
                     axis=0, keepdims=True)
        s = jnp.where(obj > _CONF_T, m, _NEG)
        if nrows < _LANES:
            s = jnp.where(lane < nrows, s, _NEG)
        xc = tt[0:1, :]
        yc = tt[1:2, :]
        w2 = tt[2:3, :] * 0.5
        h2 = tt[3:4, :] * 0.5
        y1 = yc - h2
        x1 = xc - w2
        y2 = yc + h2
        x2 = xc + w2
        s_scr[pl.ds(r, 1), :] = s
        y1_scr[pl.ds(r, 1), :] = y1
        x1_scr[pl.ds(r, 1), :] = x1
        y2_scr[pl.ds(r, 1), :] = y2
        x2_scr[pl.ds(r, 1), :] = x2
        ar_scr[pl.ds(r, 1), :] = (y2 - y1) * (x2 - x1)
        key_scr[pl.ds(r, 1), :] = (r * _LANES + lane) * 128 + ci

    eye128 = _eye(_LANES)

    def p1_body(g, _):
        for j in range(_UNROLL):
            block(g * _UNROLL + j, _LANES, eye128)
        return 0
    jax.lax.fori_loop(0, _FULL // _UNROLL, p1_body, 0)
    block(_FULL, _TAIL, _eye(_TAIL))

    ztail = jnp.zeros((_ROWS - _FULL - 1, _LANES), jnp.float32)
    s_scr[pl.ds(_FULL + 1, _ROWS - _FULL - 1), :] = ztail + _NEG
    y1_scr[pl.ds(_FULL + 1, _ROWS - _FULL - 1), :] = ztail
    x1_scr[pl.ds(_FULL + 1, _ROWS - _FULL - 1), :] = ztail
    y2_scr[pl.ds(_FULL + 1, _ROWS - _FULL - 1), :] = ztail
    x2_scr[pl.ds(_FULL + 1, _ROWS - _FULL - 1), :] = ztail
    ar_scr[pl.ds(_FULL + 1, _ROWS - _FULL - 1), :] = ztail
    key_scr[pl.ds(_FULL + 1, _ROWS - _FULL - 1), :] = ztail.astype(jnp.int32)

    def select(sv):
        keyp = key_scr[...]
        best = jnp.max(sv)
        sel = sv == best
        key = jnp.min(jnp.where(sel, keyp, _NPAD * 128))
        cnt = jnp.sum(sel.astype(jnp.float32))
        fy1 = jnp.max(jnp.where(sel, y1_scr[...], -3e38))
        fx1 = jnp.max(jnp.where(sel, x1_scr[...], -3e38))
        fy2 = jnp.max(jnp.where(sel, y2_scr[...], -3e38))
        fx2 = jnp.max(jnp.where(sel, x2_scr[...], -3e38))

        def fast(_):
            return fy1, fx1, fy2, fx2

        def slow(_):
            sel2 = keyp == key
            return (jnp.max(jnp.where(sel2, y1_scr[...], -3e38)),
                    jnp.max(jnp.where(sel2, x1_scr[...], -3e38)),
                    jnp.max(jnp.where(sel2, y2_scr[...], -3e38)),
                    jnp.max(jnp.where(sel2, x2_scr[...], -3e38)))

        by1, bx1, by2, bx2 = jax.lax.cond(cnt == 1.0, fast, slow, None)
        return best, key, by1, bx1, by2, bx2

    def body(i, carry):
        s, ok, sb, sk, s1, s2, s3, s4 = carry
        best, key, by1, bx1, by2, bx2 = jax.lax.cond(
            ok != 0.0,
            lambda _: (sb, sk, s1, s2, s3, s4),
            lambda _: select(s), None)

        keyp = key_scr[...]
        s_excl = jnp.where(keyp == key, _NEG, s)
        yy1 = jnp.maximum(y1_scr[...], by1)
        xx1 = jnp.maximum(x1_scr[...], bx1)
        yy2 = jnp.minimum(y2_scr[...], by2)
        xx2 = jnp.minimum(x2_scr[...], bx2)
        inter = (jnp.clip(yy2 - yy1, 0.0) * jnp.clip(xx2 - xx1, 0.0))
        barea = (by2 - by1) * (bx2 - bx1)
        union = ar_scr[...] + barea - inter + 1e-9
        s_next = jnp.where(inter > _IOU_T * union, _NEG, s_excl)

        nb, nk, n1, n2, n3, n4 = select(s_excl)
        qy1 = jnp.maximum(n1, by1)
        qx1 = jnp.maximum(n2, bx1)
        qy2 = jnp.minimum(n3, by2)
        qx2 = jnp.minimum(n4, bx2)
        qi = (jnp.clip(qy2 - qy1, 0.0) * jnp.clip(qx2 - qx1, 0.0))
        qa = (n3 - n1) * (n4 - n2)
        qu = qa + barea - qi + 1e-9
        ok_next = jnp.where(qi > _IOU_T * qu, 0.0, 1.0)

        idx = key >> 7
        cls = key & 127
        valid = best > _NEG * 0.5
        main = jnp.concatenate(
            [jnp.stack([by1, bx1, by2, bx2,
                        cls.astype(jnp.float32), best]).reshape(1, 6),
             jnp.zeros((1, 2), jnp.float32)], axis=1)
        ob_ref[pl.ds(i, 1), :] = jnp.where(valid, main, 0.0)
        mrow = rows_ref[0, pl.ds(idx, 1), 5 + _NC:]
        om_ref[pl.ds(i, 1), :] = jnp.where(valid, mrow, 0.0)
        return (s_next, ok_next, nb, nk, n1, n2, n3, n4)

    zf = jnp.float32(0)
    jax.lax.fori_loop(
        0, _MAXDET, body,
        (s_scr[...], zf, zf, jnp.int32(0), zf, zf, zf, zf))


@jax.jit
def kernel(predictions):
    out_shapes = (
        jax.ShapeDtypeStruct((_MAXDET, 8), jnp.float32),
        jax.ShapeDtypeStruct((_MAXDET, _MASK), jnp.float32),
    )
    main, masks = pl.pallas_call(
        _nms_body,
        out_shape=out_shapes,
        scratch_shapes=([pltpu.VMEM((_ROWS, _LANES), jnp.float32)
                         for _ in range(5)]
                        + [pltpu.VMEM((_ROWS, _LANES), jnp.int32)]
                        + [pltpu.VMEM((_ROWS, _LANES), jnp.float32)]),
    )(predictions)
    return (main[None, :, :4],
            main[:, 4].reshape(1, _MAXDET),
            main[:, 5].reshape(1, _MAXDET),
            masks[None])
